# baseline (speedup 1.0000x reference)
import functools
import jax
import jax.numpy as jnp
from jax import lax
from jax.experimental import pallas as pl
from jax.experimental.pallas import tpu as pltpu
from jax.experimental.pallas import tpu_sc as plsc

_L = 16


def _sc_body(col_hbm, row_hbm, out_hbm, rowbuf, chunk, sem, osem):
    s = lax.axis_index("s")
    c = lax.axis_index("c")
    w = s * 2 + c
    q = w % 4
    g = w // 4
    lcopies = [
        pltpu.async_copy(col_hbm, chunk.at[pl.ds(32 * k, 32), pl.ds(0, 128)], sem)
        for k in range(8)
    ]
    pltpu.sync_copy(row_hbm.at[pl.ds(q * 8, 8)], rowbuf)
    for r in range(8):
        for j in range(128 // _L):
            v = rowbuf[r, _L * j:_L * (j + 1)]
            for i in range(32):
                chunk[r * 32 + i, 128 + _L * j:128 + _L * (j + 1)] = v
    for cp in lcopies:
        cp.wait()
    ocopies = [
        pltpu.async_copy(chunk, out_hbm.at[g + 8 * k, pl.ds(q * 256, 256), :], osem)
        for k in range(4)
    ]
    for cp in ocopies:
        cp.wait()


def kernel(x, h, w, row_embed, col_embed):
    B, HW, D = x.shape
    col = jax.lax.slice(col_embed, (0, 0), (32, 128))
    row = jax.lax.slice(row_embed, (0, 0), (32, 128))
    mesh = plsc.VectorSubcoreMesh(core_axis_name="c", subcore_axis_name="s")
    k = functools.partial(
        pl.kernel,
        mesh=mesh,
        out_type=jax.ShapeDtypeStruct((B, HW, D), jnp.float32),
        scratch_types=[
            pltpu.VMEM((8, 128), jnp.float32),
            pltpu.VMEM((256, 256), jnp.float32),
            pltpu.SemaphoreType.DMA,
            pltpu.SemaphoreType.DMA,
        ],
    )(_sc_body)
    return k(col, row)


# speedup vs baseline: 1.4395x; 1.4395x over previous
import functools
import jax
import jax.numpy as jnp
from jax import lax
from jax.experimental import pallas as pl
from jax.experimental.pallas import tpu as pltpu
from jax.experimental.pallas import tpu_sc as plsc

_L = 16


def _sc_lookup_body(col_hbm, row_hbm, pos_hbm, colbuf, rowbuf, chunk, sem):
    s = lax.axis_index("s")
    c = lax.axis_index("c")
    w = s * 2 + c
    pltpu.sync_copy(col_hbm, colbuf)
    pltpu.sync_copy(row_hbm.at[w], rowbuf)
    for i in range(32):
        for j in range(128 // _L):
            chunk[i, _L * j:_L * (j + 1)] = colbuf[i, _L * j:_L * (j + 1)]
    for j in range(128 // _L):
        v = rowbuf[_L * j:_L * (j + 1)]
        for i in range(32):
            chunk[i, 128 + _L * j:128 + _L * (j + 1)] = v
    pltpu.sync_copy(chunk, pos_hbm.at[pl.ds(w * 32, 32), :])


def _tc_broadcast_body(pos_ref, out_ref):
    BB, HW, D = out_ref.shape
    out_ref[...] = jnp.broadcast_to(pos_ref[...][None], (BB, HW, D))


def kernel(x, h, w, row_embed, col_embed):
    B, HW, D = x.shape
    col = jax.lax.slice(col_embed, (0, 0), (32, 128))
    row = jax.lax.slice(row_embed, (0, 0), (32, 128))
    mesh = plsc.VectorSubcoreMesh(core_axis_name="c", subcore_axis_name="s")
    sc_lookup = functools.partial(
        pl.kernel,
        mesh=mesh,
        out_type=jax.ShapeDtypeStruct((HW, D), jnp.float32),
        scratch_types=[
            pltpu.VMEM((32, 128), jnp.float32),
            pltpu.VMEM((128,), jnp.float32),
            pltpu.VMEM((32, 256), jnp.float32),
            pltpu.SemaphoreType.DMA,
        ],
    )(_sc_lookup_body)
    pos = sc_lookup(col, row)

    BB = 4
    out = pl.pallas_call(
        _tc_broadcast_body,
        grid=(B // BB,),
        in_specs=[pl.BlockSpec((HW, D), lambda b: (0, 0))],
        out_specs=pl.BlockSpec((BB, HW, D), lambda b: (b, 0, 0)),
        out_shape=jax.ShapeDtypeStruct((B, HW, D), jnp.float32),
    )(pos)
    return out


# speedup vs baseline: 1.4535x; 1.0097x over previous
import functools
import jax
import jax.numpy as jnp
from jax import lax
from jax.experimental import pallas as pl
from jax.experimental.pallas import tpu as pltpu
from jax.experimental.pallas import tpu_sc as plsc

_L = 16
_BB = 4
_NB1 = 5


def _sc_lookup_body(col_hbm, row_hbm, pos_hbm, colbuf, rowbuf, chunk, sem):
    s = lax.axis_index("s")
    c = lax.axis_index("c")
    w = s * 2 + c
    pltpu.sync_copy(col_hbm, colbuf)
    pltpu.sync_copy(row_hbm.at[w], rowbuf)
    for i in range(32):
        for j in range(128 // _L):
            chunk[i, _L * j:_L * (j + 1)] = colbuf[i, _L * j:_L * (j + 1)]
    for j in range(128 // _L):
        v = rowbuf[_L * j:_L * (j + 1)]
        for i in range(32):
            chunk[i, 128 + _L * j:128 + _L * (j + 1)] = v
    pltpu.sync_copy(chunk, pos_hbm.at[pl.ds(w * 32, 32), :])


def _tc_build_body(col_ref, row_ref, out_ref):
    BB, HW, D = out_ref.shape
    W = col_ref.shape[0]
    H = row_ref.shape[0]
    col = col_ref[...]
    row = row_ref[...]
    left = jnp.broadcast_to(col[None, :, :], (H, W, 128)).reshape(HW, 128)
    right = jnp.broadcast_to(row[:, None, :], (H, W, 128)).reshape(HW, 128)
    pos = jnp.concatenate([left, right], axis=-1)
    out_ref[...] = jnp.broadcast_to(pos[None], (BB, HW, D))


def _tc_fill_body(prev_ref, pos_ref, out_ref):
    BB, HW, D = out_ref.shape
    out_ref[...] = jnp.broadcast_to(pos_ref[...][None], (BB, HW, D))


def kernel(x, h, w, row_embed, col_embed):
    B, HW, D = x.shape
    col = jax.lax.slice(col_embed, (0, 0), (32, 128))
    row = jax.lax.slice(row_embed, (0, 0), (32, 128))

    mesh = plsc.VectorSubcoreMesh(core_axis_name="c", subcore_axis_name="s")
    sc_lookup = functools.partial(
        pl.kernel,
        mesh=mesh,
        out_type=jax.ShapeDtypeStruct((HW, D), jnp.float32),
        scratch_types=[
            pltpu.VMEM((32, 128), jnp.float32),
            pltpu.VMEM((128,), jnp.float32),
            pltpu.VMEM((32, 256), jnp.float32),
            pltpu.SemaphoreType.DMA,
        ],
    )(_sc_lookup_body)
    pos = sc_lookup(col, row)

    out1 = pl.pallas_call(
        _tc_build_body,
        grid=(_NB1,),
        in_specs=[
            pl.BlockSpec((32, 128), lambda b: (0, 0)),
            pl.BlockSpec((32, 128), lambda b: (0, 0)),
        ],
        out_specs=pl.BlockSpec((_BB, HW, D), lambda b: (b, 0, 0)),
        out_shape=jax.ShapeDtypeStruct((B, HW, D), jnp.float32),
    )(col, row)

    out = pl.pallas_call(
        _tc_fill_body,
        grid=(B // _BB - _NB1,),
        in_specs=[
            pl.BlockSpec((1, 8, 128), lambda b: (0, 0, 0)),
            pl.BlockSpec((HW, D), lambda b: (0, 0)),
        ],
        out_specs=pl.BlockSpec((_BB, HW, D), lambda b: (b + _NB1, 0, 0)),
        out_shape=jax.ShapeDtypeStruct((B, HW, D), jnp.float32),
        input_output_aliases={0: 0},
    )(out1, pos)
    return out


# speedup vs baseline: 1.4990x; 1.0313x over previous
import functools
import jax
import jax.numpy as jnp
from jax import lax
from jax.experimental import pallas as pl
from jax.experimental.pallas import tpu as pltpu
from jax.experimental.pallas import tpu_sc as plsc

_L = 16
_BB = 4
_NB1 = 5


def _sc_lookup_body(col_hbm, row_hbm, pos_hbm, colbuf, rowbuf, chunk, sem):
    s = lax.axis_index("s")
    pltpu.sync_copy(col_hbm, colbuf)
    for half in range(2):
        w = s * 2 + half
        pltpu.sync_copy(row_hbm.at[w], rowbuf)
        for i in range(32):
            for j in range(128 // _L):
                chunk[i, _L * j:_L * (j + 1)] = colbuf[i, _L * j:_L * (j + 1)]
        for j in range(128 // _L):
            v = rowbuf[_L * j:_L * (j + 1)]
            for i in range(32):
                chunk[i, 128 + _L * j:128 + _L * (j + 1)] = v
        pltpu.sync_copy(chunk, pos_hbm.at[pl.ds(w * 32, 32), :])


def _tc_build_body(col_ref, row_ref, out_ref):
    BB, HW, D = out_ref.shape
    W = col_ref.shape[0]
    H = row_ref.shape[0]
    col = col_ref[...]
    row = row_ref[...]
    left = jnp.broadcast_to(col[None, :, :], (H, W, 128)).reshape(HW, 128)
    right = jnp.broadcast_to(row[:, None, :], (H, W, 128)).reshape(HW, 128)
    pos = jnp.concatenate([left, right], axis=-1)
    out_ref[...] = jnp.broadcast_to(pos[None], (BB, HW, D))


def _tc_fill_body(prev_ref, pos_ref, out_ref):
    BB, HW, D = out_ref.shape
    out_ref[...] = jnp.broadcast_to(pos_ref[...][None], (BB, HW, D))


def kernel(x, h, w, row_embed, col_embed):
    B, HW, D = x.shape
    col = jax.lax.slice(col_embed, (0, 0), (32, 128))
    row = jax.lax.slice(row_embed, (0, 0), (32, 128))

    mesh = plsc.VectorSubcoreMesh(
        core_axis_name="c", subcore_axis_name="s", num_cores=1
    )
    sc_lookup = functools.partial(
        pl.kernel,
        mesh=mesh,
        out_type=jax.ShapeDtypeStruct((HW, D), jnp.float32),
        scratch_types=[
            pltpu.VMEM((32, 128), jnp.float32),
            pltpu.VMEM((128,), jnp.float32),
            pltpu.VMEM((32, 256), jnp.float32),
            pltpu.SemaphoreType.DMA,
        ],
    )(_sc_lookup_body)
    pos = sc_lookup(col, row)

    out1 = pl.pallas_call(
        _tc_build_body,
        grid=(_NB1,),
        in_specs=[
            pl.BlockSpec((32, 128), lambda b: (0, 0)),
            pl.BlockSpec((32, 128), lambda b: (0, 0)),
        ],
        out_specs=pl.BlockSpec((_BB, HW, D), lambda b: (b, 0, 0)),
        out_shape=jax.ShapeDtypeStruct((B, HW, D), jnp.float32),
    )(col, row)

    out = pl.pallas_call(
        _tc_fill_body,
        grid=(B // _BB - _NB1,),
        in_specs=[
            pl.BlockSpec((1, 8, 128), lambda b: (0, 0, 0)),
            pl.BlockSpec((HW, D), lambda b: (0, 0)),
        ],
        out_specs=pl.BlockSpec((_BB, HW, D), lambda b: (b + _NB1, 0, 0)),
        out_shape=jax.ShapeDtypeStruct((B, HW, D), jnp.float32),
        input_output_aliases={0: 0},
    )(out1, pos)
    return out
